# baseline (speedup 1.0000x reference)
import functools

import jax
import jax.numpy as jnp
from jax import lax
from jax.experimental import pallas as pl
from jax.experimental.pallas import tpu as pltpu
from jax.experimental.pallas import tpu_sc as plsc

NUM_CODES = 8192
DIM = 64
ROWS = 16384
R_BLK = 256
C_BLK = 1024

SC_CORES = 2
SC_SUBCORES = 16
SC_WORKERS = SC_CORES * SC_SUBCORES
ROWS_PER_WORKER = ROWS // SC_WORKERS


def _argmin_body(x_ref, e_ref, idx_ref, loss_ref, acc_ref):
    i = pl.program_id(0)
    x = x_ref[...]
    x2 = jnp.sum(x * x, axis=1, keepdims=True)

    def chunk(j, carry):
        run_min, run_idx = carry
        e = e_ref[pl.ds(j * C_BLK, C_BLK), :]
        e2 = jnp.sum(e * e, axis=1)
        m = lax.dot_general(
            x, e, (((1,), (1,)), ((), ())),
            preferred_element_type=jnp.float32)
        d = (x2 - 2.0 * m) + e2[None, :]
        bm = jnp.min(d, axis=1, keepdims=True)
        li = lax.broadcasted_iota(jnp.int32, (R_BLK, C_BLK), 1)
        bi = jnp.min(jnp.where(d == bm, li, jnp.int32(NUM_CODES)),
                     axis=1, keepdims=True) + j * C_BLK
        better = bm < run_min
        return jnp.minimum(run_min, bm), jnp.where(better, bi, run_idx)

    init = (jnp.full((R_BLK, 1), jnp.inf, jnp.float32),
            jnp.zeros((R_BLK, 1), jnp.int32))
    run_min, run_idx = lax.fori_loop(0, NUM_CODES // C_BLK, chunk, init)
    idx_ref[...] = run_idx[:, 0]

    @pl.when(i == 0)
    def _():
        acc_ref[0] = 0.0

    acc_ref[0] += jnp.sum(run_min)

    @pl.when(i == pl.num_programs(0) - 1)
    def _():
        loss_ref[0, 0] = acc_ref[0]


def _argmin_call(flat, embedding):
    return pl.pallas_call(
        _argmin_body,
        grid=(ROWS // R_BLK,),
        in_specs=[
            pl.BlockSpec((R_BLK, DIM), lambda i: (i, 0)),
            pl.BlockSpec((NUM_CODES, DIM), lambda i: (0, 0)),
        ],
        out_specs=[
            pl.BlockSpec((R_BLK,), lambda i: (i,)),
            pl.BlockSpec(memory_space=pltpu.SMEM),
        ],
        out_shape=[
            jax.ShapeDtypeStruct((ROWS,), jnp.int32),
            jax.ShapeDtypeStruct((1, 1), jnp.float32),
        ],
        scratch_shapes=[pltpu.SMEM((1,), jnp.float32)],
    )(flat, embedding)


def _gather_call(embedding, idx_flat):
    mesh = plsc.VectorSubcoreMesh(core_axis_name="c", subcore_axis_name="s")

    @functools.partial(
        pl.kernel,
        mesh=mesh,
        compiler_params=pltpu.CompilerParams(use_tc_tiling_on_sc=False),
        out_type=jax.ShapeDtypeStruct((ROWS, DIM), jnp.float32),
        scratch_types=[
            pltpu.VMEM((ROWS_PER_WORKER,), jnp.int32),
            pltpu.VMEM((ROWS_PER_WORKER, DIM), jnp.float32),
            pltpu.SemaphoreType.DMA,
        ],
    )
    def gather(table_hbm, idx_hbm, out_hbm, idx_v, rows_v, sem):
        wid = lax.axis_index("s") * SC_CORES + lax.axis_index("c")
        base = wid * ROWS_PER_WORKER
        pltpu.sync_copy(idx_hbm.at[pl.ds(base, ROWS_PER_WORKER)], idx_v)
        pltpu.async_copy(table_hbm.at[idx_v], rows_v, sem).wait()
        pltpu.sync_copy(rows_v, out_hbm.at[pl.ds(base, ROWS_PER_WORKER)])

    return gather(embedding, idx_flat)


def kernel(z_e, embedding):
    B, D, H, W = z_e.shape
    flat = jnp.transpose(z_e, (0, 2, 3, 1)).reshape(-1, D)
    idx_flat, loss_acc = _argmin_call(flat, embedding)
    z_q_flat = _gather_call(embedding, idx_flat)
    z_q = jnp.transpose(z_q_flat.reshape(B, H, W, D), (0, 3, 1, 2))
    z_q_st = z_e + lax.stop_gradient(z_q - z_e)
    loss = loss_acc[0, 0] / (B * H * W * D)
    return (z_q_st, loss, idx_flat.reshape(B, H, W))


# speedup vs baseline: 1.3815x; 1.3815x over previous
import functools

import jax
import jax.numpy as jnp
from jax import lax
from jax.experimental import pallas as pl
from jax.experimental.pallas import tpu as pltpu
from jax.experimental.pallas import tpu_sc as plsc

NUM_CODES = 8192
DIM = 64
ROWS = 16384
R_BLK = 256
C_BLK = 1024

SC_CORES = 2
SC_SUBCORES = 16
SC_WORKERS = SC_CORES * SC_SUBCORES
ROWS_PER_WORKER = ROWS // SC_WORKERS


LANES = 128
GRPS_PER_CHUNK = C_BLK // LANES


def _argmin_body(x_ref, e_ref, idx_ref, loss_ref, e2_ref, acc_ref):
    i = pl.program_id(0)

    @pl.when(i == 0)
    def _():
        acc_ref[0] = 0.0

        def pre(j, c):
            e = e_ref[pl.ds(j * C_BLK, C_BLK), :]
            e2_ref[0, pl.ds(j * C_BLK, C_BLK)] = jnp.sum(e * e, axis=1)
            return c

        lax.fori_loop(0, NUM_CODES // C_BLK, pre, 0)

    x = x_ref[...]
    xm2 = -2.0 * x
    x2 = jnp.sum(x * x, axis=1, keepdims=True)

    def chunk(j, carry):
        run_min, run_grp = carry
        e = e_ref[pl.ds(j * C_BLK, C_BLK), :]
        m2 = lax.dot_general(
            xm2, e, (((1,), (1,)), ((), ())),
            preferred_element_type=jnp.float32)
        e2 = e2_ref[:, pl.ds(j * C_BLK, C_BLK)]
        d = (x2 + m2) + e2
        for g in range(GRPS_PER_CHUNK):
            dg = d[:, g * LANES:(g + 1) * LANES]
            better = dg < run_min
            run_min = jnp.minimum(run_min, dg)
            run_grp = jnp.where(better, j * GRPS_PER_CHUNK + g, run_grp)
        return run_min, run_grp

    init = (jnp.full((R_BLK, LANES), jnp.inf, jnp.float32),
            jnp.zeros((R_BLK, LANES), jnp.int32))
    run_min, run_grp = lax.fori_loop(0, NUM_CODES // C_BLK, chunk, init)

    best = jnp.min(run_min, axis=1, keepdims=True)
    code = run_grp * LANES + lax.broadcasted_iota(
        jnp.int32, (R_BLK, LANES), 1)
    idx = jnp.min(jnp.where(run_min == best, code, jnp.int32(NUM_CODES)),
                  axis=1)
    idx_ref[...] = idx
    acc_ref[0] += jnp.sum(best)

    @pl.when(i == pl.num_programs(0) - 1)
    def _():
        loss_ref[0, 0] = acc_ref[0]


def _argmin_call(flat, embedding):
    return pl.pallas_call(
        _argmin_body,
        grid=(ROWS // R_BLK,),
        in_specs=[
            pl.BlockSpec((R_BLK, DIM), lambda i: (i, 0)),
            pl.BlockSpec((NUM_CODES, DIM), lambda i: (0, 0)),
        ],
        out_specs=[
            pl.BlockSpec((R_BLK,), lambda i: (i,)),
            pl.BlockSpec(memory_space=pltpu.SMEM),
        ],
        out_shape=[
            jax.ShapeDtypeStruct((ROWS,), jnp.int32),
            jax.ShapeDtypeStruct((1, 1), jnp.float32),
        ],
        scratch_shapes=[pltpu.VMEM((1, NUM_CODES), jnp.float32),
                        pltpu.SMEM((1,), jnp.float32)],
    )(flat, embedding)


def _gather_call(embedding, idx_flat):
    mesh = plsc.VectorSubcoreMesh(core_axis_name="c", subcore_axis_name="s")

    @functools.partial(
        pl.kernel,
        mesh=mesh,
        compiler_params=pltpu.CompilerParams(use_tc_tiling_on_sc=False),
        out_type=jax.ShapeDtypeStruct((ROWS, DIM), jnp.float32),
        scratch_types=[
            pltpu.VMEM((ROWS_PER_WORKER,), jnp.int32),
            pltpu.VMEM((ROWS_PER_WORKER, DIM), jnp.float32),
            pltpu.SemaphoreType.DMA,
        ],
    )
    def gather(table_hbm, idx_hbm, out_hbm, idx_v, rows_v, sem):
        wid = lax.axis_index("s") * SC_CORES + lax.axis_index("c")
        base = wid * ROWS_PER_WORKER
        pltpu.sync_copy(idx_hbm.at[pl.ds(base, ROWS_PER_WORKER)], idx_v)
        pltpu.async_copy(table_hbm.at[idx_v], rows_v, sem).wait()
        pltpu.sync_copy(rows_v, out_hbm.at[pl.ds(base, ROWS_PER_WORKER)])

    return gather(embedding, idx_flat)


def kernel(z_e, embedding):
    B, D, H, W = z_e.shape
    flat = jnp.transpose(z_e, (0, 2, 3, 1)).reshape(-1, D)
    idx_flat, loss_acc = _argmin_call(flat, embedding)
    z_q_flat = _gather_call(embedding, idx_flat)
    z_q = jnp.transpose(z_q_flat.reshape(B, H, W, D), (0, 3, 1, 2))
    z_q_st = z_e + lax.stop_gradient(z_q - z_e)
    loss = loss_acc[0, 0] / (B * H * W * D)
    return (z_q_st, loss, idx_flat.reshape(B, H, W))
